# baseline (speedup 1.0000x reference)
import functools

import jax
import jax.numpy as jnp
from jax import lax
from jax.experimental import pallas as pl
from jax.experimental.pallas import tpu as pltpu
from jax.experimental.pallas import tpu_sc as plsc

_NC = 2
_NS = 16
_NW = _NC * _NS
_LANES = 16
_R = 32


def kernel(x, pe):
    B, S, D = x.shape
    pe_rows = pe[:S]
    s_per_w = S // _NW
    n_steps = s_per_w // _R
    col_chunks = D // _LANES

    mesh = plsc.VectorSubcoreMesh(core_axis_name="c", subcore_axis_name="s")

    @functools.partial(
        pl.kernel,
        mesh=mesh,
        out_type=jax.ShapeDtypeStruct((B, S, D), jnp.float32),
        scratch_types=[
            pltpu.VMEM((_R, D), jnp.float32),
            pltpu.VMEM((_R, D), jnp.float32),
        ],
    )
    def sc_add(x_hbm, pe_hbm, out_hbm, pe_v, x_v):
        wid = lax.axis_index("s") * _NC + lax.axis_index("c")
        base = wid * s_per_w

        def step(i, _):
            row0 = base + i * _R
            pltpu.sync_copy(pe_hbm.at[pl.ds(row0, _R)], pe_v)

            def per_batch(b, _):
                pltpu.sync_copy(x_hbm.at[b, pl.ds(row0, _R)], x_v)

                def add_row(r, _):
                    def add_chunk(j, _):
                        c = j * _LANES
                        x_v[r, pl.ds(c, _LANES)] = (
                            x_v[r, pl.ds(c, _LANES)] + pe_v[r, pl.ds(c, _LANES)]
                        )
                        return 0

                    lax.fori_loop(0, col_chunks, add_chunk, 0)
                    return 0

                lax.fori_loop(0, _R, add_row, 0)
                pltpu.sync_copy(x_v, out_hbm.at[b, pl.ds(row0, _R)])
                return 0

            lax.fori_loop(0, B, per_batch, 0)
            return 0

        lax.fori_loop(0, n_steps, step, 0)

    return sc_add(x, pe_rows)


# speedup vs baseline: 1.0127x; 1.0127x over previous
import functools

import jax
import jax.numpy as jnp
from jax import lax
from jax.experimental import pallas as pl
from jax.experimental.pallas import tpu as pltpu
from jax.experimental.pallas import tpu_sc as plsc

_NC = 2
_NS = 16
_NW = _NC * _NS
_LANES = 16
_R = 16
_U = 8


def kernel(x, pe):
    B, S, D = x.shape
    pe_rows = pe[:S]
    s_per_w = S // _NW
    n_steps = s_per_w // _R
    inner_iters = D // (_U * _LANES)

    mesh = plsc.VectorSubcoreMesh(core_axis_name="c", subcore_axis_name="s")

    @functools.partial(
        pl.kernel,
        mesh=mesh,
        out_type=jax.ShapeDtypeStruct((B, S, D), jnp.float32),
        scratch_types=(
            [pltpu.VMEM((_R, D), jnp.float32)]
            + [pltpu.VMEM((_R, D), jnp.float32) for _ in range(B)]
            + [pltpu.SemaphoreType.DMA]
        ),
    )
    def sc_add(x_hbm, pe_hbm, out_hbm, pe_v, x0, x1, x2, x3, sem):
        xb = [x0, x1, x2, x3]
        wid = lax.axis_index("s") * _NC + lax.axis_index("c")
        base = wid * s_per_w

        def step(i, _):
            row0 = base + i * _R
            copies = [pltpu.async_copy(pe_hbm.at[pl.ds(row0, _R)], pe_v, sem)]
            for b in range(B):
                copies.append(
                    pltpu.async_copy(x_hbm.at[b, pl.ds(row0, _R)], xb[b], sem)
                )
            for c in copies:
                c.wait()

            def add_row(r, _):
                def add_chunks(j, _):
                    for k in range(_U):
                        c0 = (j * _U + k) * _LANES
                        pchunk = pe_v[r, pl.ds(c0, _LANES)]
                        for b in range(B):
                            xb[b][r, pl.ds(c0, _LANES)] = (
                                xb[b][r, pl.ds(c0, _LANES)] + pchunk
                            )
                    return 0

                lax.fori_loop(0, inner_iters, add_chunks, 0)
                return 0

            lax.fori_loop(0, _R, add_row, 0)
            for b in range(B):
                pltpu.sync_copy(xb[b], out_hbm.at[b, pl.ds(row0, _R)])
            return 0

        lax.fori_loop(0, n_steps, step, 0)

    return sc_add(x, pe_rows)


# speedup vs baseline: 1.1360x; 1.1218x over previous
import functools

import jax
import jax.numpy as jnp
from jax import lax
from jax.experimental import pallas as pl
from jax.experimental.pallas import tpu as pltpu
from jax.experimental.pallas import tpu_sc as plsc

_NC = 2
_NS = 16
_NW = _NC * _NS
_LANES = 16
_E = 16 * 1024
_U = 4


def kernel(x, pe):
    B, S, D = x.shape
    xf = x.reshape(B, S * D)
    pef = pe[:S].reshape(S * D)
    per_w = (S * D) // _NW
    n_steps = per_w // _E
    chunks = _E // _LANES

    mesh = plsc.VectorSubcoreMesh(core_axis_name="c", subcore_axis_name="s")

    @functools.partial(
        pl.kernel,
        mesh=mesh,
        out_type=jax.ShapeDtypeStruct((B, S * D), jnp.float32),
        scratch_types=(
            [pltpu.VMEM((_E,), jnp.float32)]
            + [pltpu.VMEM((_E,), jnp.float32) for _ in range(B)]
            + [pltpu.SemaphoreType.DMA]
        ),
    )
    def sc_add(x_hbm, pe_hbm, out_hbm, pe_v, x0, x1, x2, x3, sem):
        xb = [x0, x1, x2, x3]
        wid = lax.axis_index("s") * _NC + lax.axis_index("c")
        base = wid * per_w

        def step(i, _):
            off = base + i * _E
            copies = [pltpu.async_copy(pe_hbm.at[pl.ds(off, _E)], pe_v, sem)]
            for b in range(B):
                copies.append(
                    pltpu.async_copy(x_hbm.at[b, pl.ds(off, _E)], xb[b], sem)
                )
            for c in copies:
                c.wait()

            @plsc.parallel_loop(0, chunks, unroll=_U)
            def add_chunk(j):
                c0 = j * _LANES
                pchunk = pe_v[pl.ds(c0, _LANES)]
                for b in range(B):
                    xb[b][pl.ds(c0, _LANES)] = xb[b][pl.ds(c0, _LANES)] + pchunk

            for b in range(B):
                pltpu.sync_copy(xb[b], out_hbm.at[b, pl.ds(off, _E)])
            return 0

        lax.fori_loop(0, n_steps, step, 0)

    return sc_add(xf, pef).reshape(B, S, D)
